# baseline (speedup 1.0000x reference)
import functools

import jax
import jax.numpy as jnp
from jax import lax
from jax.experimental import pallas as pl
from jax.experimental.pallas import tpu as pltpu
from jax.experimental.pallas import tpu_sc as plsc

N = 10000
E = 320000
IN_DIM = 128
HID = 64
OUT_DIM = 128
G = 64

NC = 2
NS = 16
NW = NC * NS
L = 16
C = 128

N_PAD = 10240
RPT = N_PAD // NS
E_PAD = ((E + NW * C * 8 - 1) // (NW * C * 8)) * (NW * C * 8)
NCH = E_PAD // (NW * C)

_mesh = plsc.VectorSubcoreMesh(core_axis_name="c", subcore_axis_name="s")
_sc_params = pltpu.CompilerParams(needs_layout_passes=False,
                                  use_tc_tiling_on_sc=False)


def _deg_body(dst_hbm, ew_hbm, deg_out, dst_v, ew_v, z_v, deg_sp, sem):
    c = lax.axis_index("c")
    s = lax.axis_index("s")
    w = c * NS + s

    def zb(k, carry):
        z_v[pl.ds(k * L, L)] = jnp.zeros((L,), jnp.float32)
        return carry

    lax.fori_loop(0, RPT // L, zb, 0)
    pltpu.sync_copy(z_v, deg_sp.at[pl.ds(s * RPT, RPT)])
    plsc.subcore_barrier()

    pltpu.sync_copy(dst_hbm.at[pl.ds(w * NCH, NCH)], dst_v)
    pltpu.sync_copy(ew_hbm.at[pl.ds(w * NCH, NCH)], ew_v)

    def chunk(j, carry):
        pltpu.sync_copy(ew_v.at[j], deg_sp.at[dst_v.at[j]], add=True)
        return carry

    lax.fori_loop(0, NCH, chunk, 0)
    plsc.subcore_barrier()
    pltpu.sync_copy(deg_sp.at[pl.ds(s * RPT, RPT)],
                    deg_out.at[c, pl.ds(s * RPT, RPT)])


_deg_call = pl.kernel(
    _deg_body,
    out_type=jax.ShapeDtypeStruct((NC, N_PAD), jnp.float32),
    mesh=_mesh,
    compiler_params=_sc_params,
    scratch_types=[
        pltpu.VMEM((NCH, C), jnp.int32),
        pltpu.VMEM((NCH, C), jnp.float32),
        pltpu.VMEM((RPT,), jnp.float32),
        pltpu.VMEM_SHARED((N_PAD,), jnp.float32),
        pltpu.SemaphoreType.DMA,
    ],
)


NBUF = 4
NIX = 8


def _agg_body(g_hbm, sd_hbm, ew_hbm, zeros_hbm, out_hbm,
              r0, r1, r2, r3,
              x0, x1, x2, x3, x4, x5, x6, x7,
              e0, e1, e2, e3,
              acc_sp, g_sp,
              sg0, sg1, sg2, sg3, ss0, ss1, ss2, ss3,
              sx0, sx1, sx2, sx3, sx4, sx5, sx6, sx7,
              se0, se1, se2, se3):
    rows = (r0, r1, r2, r3)
    ixb = (x0, x1, x2, x3, x4, x5, x6, x7)
    ewb = (e0, e1, e2, e3)
    sem_g = (sg0, sg1, sg2, sg3)
    sem_s = (ss0, ss1, ss2, ss3)
    sem_x = (sx0, sx1, sx2, sx3, sx4, sx5, sx6, sx7)
    sem_e = (se0, se1, se2, se3)
    c = lax.axis_index("c")
    s = lax.axis_index("s")
    w = c * NS + s

    pltpu.sync_copy(zeros_hbm.at[pl.ds(s * RPT, RPT)],
                    acc_sp.at[pl.ds(s * RPT, RPT)])
    pltpu.sync_copy(g_hbm.at[pl.ds(s * RPT, RPT)], g_sp.at[pl.ds(s * RPT, RPT)])
    plsc.subcore_barrier()

    def idx_start(j, sl):
        b8, b4 = sl % NIX, sl % NBUF
        col = (w * NCH + j) * C
        pltpu.async_copy(sd_hbm.at[0, pl.ds(col, C)], ixb[b8].at[0], sem_x[b8])
        pltpu.async_copy(sd_hbm.at[1, pl.ds(col, C)], ixb[b8].at[1], sem_x[b8])
        pltpu.async_copy(ew_hbm.at[pl.ds(col, C)], ewb[b4], sem_e[b4])

    def idx_wait(j, sl):
        b8 = sl % NIX
        col = (w * NCH + j) * C
        pltpu.make_async_copy(sd_hbm.at[0, pl.ds(col, C)], ixb[b8].at[0],
                              sem_x[b8]).wait()
        pltpu.make_async_copy(sd_hbm.at[1, pl.ds(col, C)], ixb[b8].at[1],
                              sem_x[b8]).wait()

    def ew_wait(j, sl):
        b4 = sl % NBUF
        pltpu.make_async_copy(ew_hbm.at[pl.ds((w * NCH + j) * C, C)],
                              ewb[b4], sem_e[b4]).wait()

    def gather_start(sl):
        pltpu.async_copy(g_sp.at[ixb[sl % NIX].at[0]], rows[sl % NBUF],
                         sem_g[sl % NBUF])

    def gather_wait(sl):
        pltpu.make_async_copy(g_sp.at[ixb[sl % NIX].at[0]], rows[sl % NBUF],
                              sem_g[sl % NBUF]).wait()

    def scatter_start(sl):
        pltpu.async_copy(rows[sl % NBUF], acc_sp.at[ixb[sl % NIX].at[1]],
                         sem_s[sl % NBUF], add=True)

    def scatter_wait(sl):
        pltpu.make_async_copy(rows[sl % NBUF], acc_sp.at[ixb[sl % NIX].at[1]],
                              sem_s[sl % NBUF]).wait()

    def scale(sl):
        r = rows[sl % NBUF]
        ewv = ewb[sl % NBUF]

        @plsc.parallel_loop(0, C // L, unroll=2)
        def _(blk):
            e0 = blk * L
            ew16 = ewv[pl.ds(e0, L)]
            for i in range(L):
                bc = lax.gather(
                    ew16, jnp.full((L, 1), i, jnp.int32),
                    lax.GatherDimensionNumbers(offset_dims=(),
                                               collapsed_slice_dims=(0,),
                                               start_index_map=(0,)),
                    (1,), mode=lax.GatherScatterMode.PROMISE_IN_BOUNDS)
                for q in range(HID // L):
                    r[e0 + i, pl.ds(q * L, L)] = r[e0 + i, pl.ds(q * L, L)] * bc

    def slot(j, sl, skip_sw=False, guard=False):
        gather_wait(sl)
        ew_wait(j, sl)
        scale(sl)
        scatter_start(sl)
        if not guard:
            if not skip_sw:
                scatter_wait(sl - 2)
            idx_wait(j + 2, sl + 2)
            gather_start(sl + 2)
            idx_start(j + 4, sl + 4)
        else:
            @pl.when(j + 2 < NCH)
            def _():
                scatter_wait(sl - 2)
                idx_wait(j + 2, sl + 2)
                gather_start(sl + 2)

            @pl.when(j + 4 < NCH)
            def _():
                idx_start(j + 4, sl + 4)

    for j in range(NBUF):
        idx_start(j, j)
    for j in range(2):
        idx_wait(j, j)
        gather_start(j)
    for j in range(NIX):
        slot(j, j, skip_sw=(j < 2))

    def outer(t, carry):
        for sl in range(NIX):
            slot(NIX * t + sl, sl, guard=True)
        return carry

    lax.fori_loop(1, NCH // NIX, outer, 0)
    for sl in range(NCH - 4, NCH):
        scatter_wait(sl)

    plsc.subcore_barrier()
    pltpu.sync_copy(acc_sp.at[pl.ds(s * RPT, RPT)],
                    out_hbm.at[c, pl.ds(s * RPT, RPT)])


_agg_call = pl.kernel(
    _agg_body,
    out_type=jax.ShapeDtypeStruct((NC, N_PAD, HID), jnp.float32),
    mesh=_mesh,
    compiler_params=_sc_params,
    scratch_types=(
        [pltpu.VMEM((C, HID), jnp.float32)] * NBUF
        + [pltpu.VMEM((2, C), jnp.int32)] * NIX
        + [pltpu.VMEM((C,), jnp.float32)] * NBUF
        + [pltpu.VMEM_SHARED((N_PAD, HID), jnp.float32)] * 2
        + [pltpu.SemaphoreType.DMA] * (NBUF + NBUF + NIX + NBUF)
    ),
)


def _tc_pre_body(deg_ref, x_ref, w1_ref, g1_ref, dis_ref):
    deg = deg_ref[:, 0:1] + deg_ref[:, 1:2] + 1.0
    dis = lax.rsqrt(deg)
    h = jnp.dot(x_ref[...], w1_ref[...], preferred_element_type=jnp.float32)
    g1_ref[...] = h * dis
    dis_ref[...] = dis


_tc_pre = pl.pallas_call(
    _tc_pre_body,
    out_shape=(jax.ShapeDtypeStruct((N_PAD, HID), jnp.float32),
               jax.ShapeDtypeStruct((N_PAD, 1), jnp.float32)),
)


def _tc_mid_body(acc_ref, g1_ref, dis_ref, b1_ref, g2_ref):
    a = acc_ref[0] + acc_ref[1] + g1_ref[...]
    h = jnp.maximum(a * dis_ref[...] + b1_ref[...], 0.0)
    g2_ref[...] = h * dis_ref[...]


_tc_mid = pl.pallas_call(
    _tc_mid_body,
    out_shape=jax.ShapeDtypeStruct((N_PAD, HID), jnp.float32),
)


def _tc_post_body(acc_ref, g2_ref, dis_ref, w2_ref, b2_ref, batch_ref, out_ref):
    a = (acc_ref[0] + acc_ref[1] + g2_ref[...]) * dis_ref[...]
    y = jnp.dot(a, w2_ref[...], preferred_element_type=jnp.float32)
    y = jnp.maximum(y + b2_ref[...], 0.0)
    gid = lax.broadcasted_iota(jnp.int32, (G, N_PAD), 0)
    onehot = (batch_ref[...] == gid).astype(jnp.float32)
    sums = jnp.dot(onehot, y, preferred_element_type=jnp.float32)
    cnts = jnp.sum(onehot, axis=1, keepdims=True)
    out_ref[...] = sums / jnp.maximum(cnts, 1.0)


_tc_post = pl.pallas_call(
    _tc_post_body,
    out_shape=jax.ShapeDtypeStruct((G, OUT_DIM), jnp.float32),
)


def kernel(x, edge_index, edge_weight, batch_vec, W1, b1, W2, b2):
    sd2 = jnp.pad(edge_index.astype(jnp.int32), ((0, 0), (0, E_PAD - E)))
    dst2 = sd2[1].reshape(E_PAD // C, C)
    ew = jnp.pad(edge_weight.astype(jnp.float32), (0, E_PAD - E))
    ew2 = ew.reshape(E_PAD // C, C)

    x_p = jnp.pad(x, ((0, N_PAD - N), (0, 0)))
    batch_p = jnp.pad(batch_vec.astype(jnp.int32), (0, N_PAD - N),
                      constant_values=G).reshape(1, N_PAD)
    zeros_nd = jnp.zeros((N_PAD, HID), jnp.float32)

    deg2 = _deg_call(dst2, ew2)
    g1, dis = _tc_pre(deg2.T, x_p, W1)
    acc1 = _agg_call(g1, sd2, ew, zeros_nd)
    g2 = _tc_mid(acc1, g1, dis, b1.reshape(1, HID))
    acc2 = _agg_call(g2, sd2, ew, zeros_nd)
    return _tc_post(acc2, g2, dis, W2, b2.reshape(1, OUT_DIM), batch_p)


# speedup vs baseline: 1.0255x; 1.0255x over previous
import functools

import jax
import jax.numpy as jnp
from jax import lax
from jax.experimental import pallas as pl
from jax.experimental.pallas import tpu as pltpu
from jax.experimental.pallas import tpu_sc as plsc

N = 10000
E = 320000
IN_DIM = 128
HID = 64
OUT_DIM = 128
G = 64

NC = 2
NS = 16
NW = NC * NS
L = 16
C = 128

N_PAD = 10240
RPT = N_PAD // NS
E_PAD = ((E + NW * C * 8 - 1) // (NW * C * 8)) * (NW * C * 8)
NCH = E_PAD // (NW * C)

_mesh = plsc.VectorSubcoreMesh(core_axis_name="c", subcore_axis_name="s")
_sc_params = pltpu.CompilerParams(needs_layout_passes=False,
                                  use_tc_tiling_on_sc=False)


def _deg_body(dst_hbm, ew_hbm, deg_out, dst_v, ew_v, z_v, deg_sp, sem):
    c = lax.axis_index("c")
    s = lax.axis_index("s")
    w = c * NS + s

    def zb(k, carry):
        z_v[pl.ds(k * L, L)] = jnp.zeros((L,), jnp.float32)
        return carry

    lax.fori_loop(0, RPT // L, zb, 0)
    pltpu.sync_copy(z_v, deg_sp.at[pl.ds(s * RPT, RPT)])
    plsc.subcore_barrier()

    pltpu.sync_copy(dst_hbm.at[pl.ds(w * NCH, NCH)], dst_v)
    pltpu.sync_copy(ew_hbm.at[pl.ds(w * NCH, NCH)], ew_v)

    def chunk(j, carry):
        pltpu.sync_copy(ew_v.at[j], deg_sp.at[dst_v.at[j]], add=True)
        return carry

    lax.fori_loop(0, NCH, chunk, 0)
    plsc.subcore_barrier()
    pltpu.sync_copy(deg_sp.at[pl.ds(s * RPT, RPT)],
                    deg_out.at[c, pl.ds(s * RPT, RPT)])


_deg_call = pl.kernel(
    _deg_body,
    out_type=jax.ShapeDtypeStruct((NC, N_PAD), jnp.float32),
    mesh=_mesh,
    compiler_params=_sc_params,
    scratch_types=[
        pltpu.VMEM((NCH, C), jnp.int32),
        pltpu.VMEM((NCH, C), jnp.float32),
        pltpu.VMEM((RPT,), jnp.float32),
        pltpu.VMEM_SHARED((N_PAD,), jnp.float32),
        pltpu.SemaphoreType.DMA,
    ],
)


NBUF = 4
NIX = 8


def _agg_body(g_hbm, sd_hbm, ew_hbm, zeros_hbm, out_hbm,
              r0, r1, r2, r3,
              x0, x1, x2, x3, x4, x5, x6, x7,
              e0, e1, e2, e3,
              acc_sp, g_sp,
              sg0, sg1, sg2, sg3, ss0, ss1, ss2, ss3,
              sx0, sx1, sx2, sx3, sx4, sx5, sx6, sx7,
              se0, se1, se2, se3):
    rows = (r0, r1, r2, r3)
    ixb = (x0, x1, x2, x3, x4, x5, x6, x7)
    ewb = (e0, e1, e2, e3)
    sem_g = (sg0, sg1, sg2, sg3)
    sem_s = (ss0, ss1, ss2, ss3)
    sem_x = (sx0, sx1, sx2, sx3, sx4, sx5, sx6, sx7)
    sem_e = (se0, se1, se2, se3)
    c = lax.axis_index("c")
    s = lax.axis_index("s")
    w = c * NS + s

    pltpu.sync_copy(zeros_hbm.at[pl.ds(s * RPT, RPT)],
                    acc_sp.at[pl.ds(s * RPT, RPT)])
    pltpu.sync_copy(g_hbm.at[pl.ds(s * RPT, RPT)], g_sp.at[pl.ds(s * RPT, RPT)])
    plsc.subcore_barrier()

    def idx_start(j, sl):
        b8, b4 = sl % NIX, sl % NBUF
        col = (w * NCH + j) * C
        pltpu.async_copy(sd_hbm.at[:, pl.ds(col, C)], ixb[b8], sem_x[b8])
        pltpu.async_copy(ew_hbm.at[pl.ds(col, C)], ewb[b4], sem_e[b4])

    def idx_wait(j, sl):
        b8 = sl % NIX
        col = (w * NCH + j) * C
        pltpu.make_async_copy(sd_hbm.at[:, pl.ds(col, C)], ixb[b8],
                              sem_x[b8]).wait()

    def ew_wait(j, sl):
        b4 = sl % NBUF
        pltpu.make_async_copy(ew_hbm.at[pl.ds((w * NCH + j) * C, C)],
                              ewb[b4], sem_e[b4]).wait()

    def gather_start(sl):
        pltpu.async_copy(g_sp.at[ixb[sl % NIX].at[0]], rows[sl % NBUF],
                         sem_g[sl % NBUF])

    def gather_wait(sl):
        pltpu.make_async_copy(g_sp.at[ixb[sl % NIX].at[0]], rows[sl % NBUF],
                              sem_g[sl % NBUF]).wait()

    def scatter_start(sl):
        pltpu.async_copy(rows[sl % NBUF], acc_sp.at[ixb[sl % NIX].at[1]],
                         sem_s[sl % NBUF], add=True)

    def scatter_wait(sl):
        pltpu.make_async_copy(rows[sl % NBUF], acc_sp.at[ixb[sl % NIX].at[1]],
                              sem_s[sl % NBUF]).wait()

    def scale(sl):
        r = rows[sl % NBUF]
        ew = ewb[sl % NBUF]

        @plsc.parallel_loop(0, C, unroll=4)
        def _(e):
            bc = plsc.load_gather(ew, [jnp.full((L,), e, jnp.int32)])
            for q in range(HID // L):
                r[e, pl.ds(q * L, L)] = r[e, pl.ds(q * L, L)] * bc

    def slot(j, sl, skip_sw=False, guard=False):
        gather_wait(sl)
        ew_wait(j, sl)
        scale(sl)
        scatter_start(sl)
        if not guard:
            if not skip_sw:
                scatter_wait(sl - 2)
            idx_wait(j + 2, sl + 2)
            gather_start(sl + 2)
            idx_start(j + 4, sl + 4)
        else:
            @pl.when(j + 2 < NCH)
            def _():
                scatter_wait(sl - 2)
                idx_wait(j + 2, sl + 2)
                gather_start(sl + 2)

            @pl.when(j + 4 < NCH)
            def _():
                idx_start(j + 4, sl + 4)

    for j in range(NBUF):
        idx_start(j, j)
    for j in range(2):
        idx_wait(j, j)
        gather_start(j)
    for j in range(NIX):
        slot(j, j, skip_sw=(j < 2))

    def outer(t, carry):
        for sl in range(NIX):
            slot(NIX * t + sl, sl, guard=True)
        return carry

    lax.fori_loop(1, NCH // NIX, outer, 0)
    for sl in range(NCH - 4, NCH):
        scatter_wait(sl)

    plsc.subcore_barrier()
    pltpu.sync_copy(acc_sp.at[pl.ds(s * RPT, RPT)],
                    out_hbm.at[c, pl.ds(s * RPT, RPT)])


_agg_call = pl.kernel(
    _agg_body,
    out_type=jax.ShapeDtypeStruct((NC, N_PAD, HID), jnp.float32),
    mesh=_mesh,
    compiler_params=_sc_params,
    scratch_types=(
        [pltpu.VMEM((C, HID), jnp.float32)] * NBUF
        + [pltpu.VMEM((2, C), jnp.int32)] * NIX
        + [pltpu.VMEM((C,), jnp.float32)] * NBUF
        + [pltpu.VMEM_SHARED((N_PAD, HID), jnp.float32)] * 2
        + [pltpu.SemaphoreType.DMA] * (NBUF + NBUF + NIX + NBUF)
    ),
)


def _tc_pre_body(deg_ref, x_ref, w1_ref, g1_ref, dis_ref):
    deg = deg_ref[:, 0:1] + deg_ref[:, 1:2] + 1.0
    dis = lax.rsqrt(deg)
    h = jnp.dot(x_ref[...], w1_ref[...], preferred_element_type=jnp.float32)
    g1_ref[...] = h * dis
    dis_ref[...] = dis


_tc_pre = pl.pallas_call(
    _tc_pre_body,
    out_shape=(jax.ShapeDtypeStruct((N_PAD, HID), jnp.float32),
               jax.ShapeDtypeStruct((N_PAD, 1), jnp.float32)),
)


def _tc_mid_body(acc_ref, g1_ref, dis_ref, b1_ref, g2_ref):
    a = acc_ref[0] + acc_ref[1] + g1_ref[...]
    h = jnp.maximum(a * dis_ref[...] + b1_ref[...], 0.0)
    g2_ref[...] = h * dis_ref[...]


_tc_mid = pl.pallas_call(
    _tc_mid_body,
    out_shape=jax.ShapeDtypeStruct((N_PAD, HID), jnp.float32),
)


def _tc_post_body(acc_ref, g2_ref, dis_ref, w2_ref, b2_ref, batch_ref, out_ref):
    a = (acc_ref[0] + acc_ref[1] + g2_ref[...]) * dis_ref[...]
    y = jnp.dot(a, w2_ref[...], preferred_element_type=jnp.float32)
    y = jnp.maximum(y + b2_ref[...], 0.0)
    gid = lax.broadcasted_iota(jnp.int32, (G, N_PAD), 0)
    onehot = (batch_ref[...] == gid).astype(jnp.float32)
    sums = jnp.dot(onehot, y, preferred_element_type=jnp.float32)
    cnts = jnp.sum(onehot, axis=1, keepdims=True)
    out_ref[...] = sums / jnp.maximum(cnts, 1.0)


_tc_post = pl.pallas_call(
    _tc_post_body,
    out_shape=jax.ShapeDtypeStruct((G, OUT_DIM), jnp.float32),
)


def kernel(x, edge_index, edge_weight, batch_vec, W1, b1, W2, b2):
    sd2 = jnp.pad(edge_index.astype(jnp.int32), ((0, 0), (0, E_PAD - E)))
    dst2 = sd2[1].reshape(E_PAD // C, C)
    ew = jnp.pad(edge_weight.astype(jnp.float32), (0, E_PAD - E))
    ew2 = ew.reshape(E_PAD // C, C)

    x_p = jnp.pad(x, ((0, N_PAD - N), (0, 0)))
    batch_p = jnp.pad(batch_vec.astype(jnp.int32), (0, N_PAD - N),
                      constant_values=G).reshape(1, N_PAD)
    zeros_nd = jnp.zeros((N_PAD, HID), jnp.float32)

    deg2 = _deg_call(dst2, ew2)
    g1, dis = _tc_pre(deg2.T, x_p, W1)
    acc1 = _agg_call(g1, sd2, ew, zeros_nd)
    g2 = _tc_mid(acc1, g1, dis, b1.reshape(1, HID))
    acc2 = _agg_call(g2, sd2, ew, zeros_nd)
    return _tc_post(acc2, g2, dis, W2, b2.reshape(1, OUT_DIM), batch_p)


# speedup vs baseline: 1.0288x; 1.0033x over previous
import functools

import jax
import jax.numpy as jnp
from jax import lax
from jax.experimental import pallas as pl
from jax.experimental.pallas import tpu as pltpu
from jax.experimental.pallas import tpu_sc as plsc

N = 10000
E = 320000
IN_DIM = 128
HID = 64
OUT_DIM = 128
G = 64

NC = 2
NS = 16
NW = NC * NS
L = 16
C = 128

N_PAD = 10240
RPT = N_PAD // NS
E_PAD = ((E + NW * C * 8 - 1) // (NW * C * 8)) * (NW * C * 8)
NCH = E_PAD // (NW * C)

_mesh = plsc.VectorSubcoreMesh(core_axis_name="c", subcore_axis_name="s")
_sc_params = pltpu.CompilerParams(needs_layout_passes=False,
                                  use_tc_tiling_on_sc=False)


def _deg_body(dst_hbm, ew_hbm, deg_out, dst_v, ew_v, z_v, deg_sp, sem):
    c = lax.axis_index("c")
    s = lax.axis_index("s")
    w = c * NS + s

    def zb(k, carry):
        z_v[pl.ds(k * L, L)] = jnp.zeros((L,), jnp.float32)
        return carry

    lax.fori_loop(0, RPT // L, zb, 0)
    pltpu.sync_copy(z_v, deg_sp.at[pl.ds(s * RPT, RPT)])
    plsc.subcore_barrier()

    pltpu.sync_copy(dst_hbm.at[pl.ds(w * NCH, NCH)], dst_v)
    pltpu.sync_copy(ew_hbm.at[pl.ds(w * NCH, NCH)], ew_v)

    def chunk(j, carry):
        pltpu.sync_copy(ew_v.at[j], deg_sp.at[dst_v.at[j]], add=True)
        return carry

    lax.fori_loop(0, NCH, chunk, 0)
    plsc.subcore_barrier()
    pltpu.sync_copy(deg_sp.at[pl.ds(s * RPT, RPT)],
                    deg_out.at[c, pl.ds(s * RPT, RPT)])


_deg_call = pl.kernel(
    _deg_body,
    out_type=jax.ShapeDtypeStruct((NC, N_PAD), jnp.float32),
    mesh=_mesh,
    compiler_params=_sc_params,
    scratch_types=[
        pltpu.VMEM((NCH, C), jnp.int32),
        pltpu.VMEM((NCH, C), jnp.float32),
        pltpu.VMEM((RPT,), jnp.float32),
        pltpu.VMEM_SHARED((N_PAD,), jnp.float32),
        pltpu.SemaphoreType.DMA,
    ],
)


NBUF = 4
NIX = 8


def _agg_body(g_hbm, sd_hbm, ew_hbm, zeros_hbm, out_hbm,
              r0, r1, r2, r3,
              x0, x1, x2, x3, x4, x5, x6, x7,
              e0, e1, e2, e3,
              acc_sp, g_sp,
              sg0, sg1, sg2, sg3, ss0, ss1, ss2, ss3,
              sx0, sx1, sx2, sx3, sx4, sx5, sx6, sx7,
              se0, se1, se2, se3):
    rows = (r0, r1, r2, r3)
    ixb = (x0, x1, x2, x3, x4, x5, x6, x7)
    ewb = (e0, e1, e2, e3)
    sem_g = (sg0, sg1, sg2, sg3)
    sem_s = (ss0, ss1, ss2, ss3)
    sem_x = (sx0, sx1, sx2, sx3, sx4, sx5, sx6, sx7)
    sem_e = (se0, se1, se2, se3)
    c = lax.axis_index("c")
    s = lax.axis_index("s")
    w = c * NS + s

    pltpu.sync_copy(zeros_hbm.at[pl.ds(s * RPT, RPT)],
                    acc_sp.at[pl.ds(s * RPT, RPT)])
    pltpu.sync_copy(g_hbm.at[pl.ds(s * RPT, RPT)], g_sp.at[pl.ds(s * RPT, RPT)])
    plsc.subcore_barrier()

    def idx_start(j, sl):
        b8, b4 = sl % NIX, sl % NBUF
        col = (w * NCH + j) * C
        pltpu.async_copy(sd_hbm.at[:, pl.ds(col, C)], ixb[b8], sem_x[b8])
        pltpu.async_copy(ew_hbm.at[pl.ds(col, C)], ewb[b4], sem_e[b4])

    def idx_wait(j, sl):
        b8 = sl % NIX
        col = (w * NCH + j) * C
        pltpu.make_async_copy(sd_hbm.at[:, pl.ds(col, C)], ixb[b8],
                              sem_x[b8]).wait()

    def ew_wait(j, sl):
        b4 = sl % NBUF
        pltpu.make_async_copy(ew_hbm.at[pl.ds((w * NCH + j) * C, C)],
                              ewb[b4], sem_e[b4]).wait()

    def gather_start(sl):
        pltpu.async_copy(g_sp.at[ixb[sl % NIX].at[0]], rows[sl % NBUF],
                         sem_g[sl % NBUF])

    def gather_wait(sl):
        pltpu.make_async_copy(g_sp.at[ixb[sl % NIX].at[0]], rows[sl % NBUF],
                              sem_g[sl % NBUF]).wait()

    def scatter_start(sl):
        pltpu.async_copy(rows[sl % NBUF], acc_sp.at[ixb[sl % NIX].at[1]],
                         sem_s[sl % NBUF], add=True)

    def scatter_wait(sl):
        pltpu.make_async_copy(rows[sl % NBUF], acc_sp.at[ixb[sl % NIX].at[1]],
                              sem_s[sl % NBUF]).wait()

    def scale(sl):
        r = rows[sl % NBUF]
        ew = ewb[sl % NBUF]

        @plsc.parallel_loop(0, C, unroll=4)
        def _(e):
            bc = plsc.load_gather(ew, [jnp.full((L,), e, jnp.int32)])
            for q in range(HID // L):
                r[e, pl.ds(q * L, L)] = r[e, pl.ds(q * L, L)] * bc

    def slot(j, sl, skip_sw=False, guard=False):
        gather_wait(sl)
        ew_wait(j, sl)
        scale(sl)
        scatter_start(sl)
        if not guard:
            if not skip_sw:
                scatter_wait(sl - 2)
            idx_wait(j + 2, sl + 2)
            gather_start(sl + 2)
            idx_start(j + 4, sl + 4)
        else:
            @pl.when(j + 2 < NCH)
            def _():
                scatter_wait(sl - 2)
                idx_wait(j + 2, sl + 2)
                gather_start(sl + 2)

            @pl.when(j + 4 < NCH)
            def _():
                idx_start(j + 4, sl + 4)

    for j in range(NBUF):
        idx_start(j, j)
    for j in range(2):
        idx_wait(j, j)
        gather_start(j)
    for j in range(NIX):
        slot(j, j, skip_sw=(j < 2))

    def outer(t, carry):
        for sl in range(NIX):
            slot(NIX * t + sl, sl, guard=True)
        return carry

    lax.fori_loop(1, NCH // NIX, outer, 0)
    for sl in range(NCH - 4, NCH):
        scatter_wait(sl)

    plsc.subcore_barrier()
    pltpu.sync_copy(acc_sp.at[pl.ds(s * RPT, RPT)],
                    out_hbm.at[c, pl.ds(s * RPT, RPT)])


_agg_call = pl.kernel(
    _agg_body,
    out_type=jax.ShapeDtypeStruct((NC, N_PAD, HID), jnp.float32),
    mesh=_mesh,
    compiler_params=_sc_params,
    scratch_types=(
        [pltpu.VMEM((C, HID), jnp.float32)] * NBUF
        + [pltpu.VMEM((2, C), jnp.int32)] * NIX
        + [pltpu.VMEM((C,), jnp.float32)] * NBUF
        + [pltpu.VMEM_SHARED((N_PAD, HID), jnp.float32)] * 2
        + [pltpu.SemaphoreType.DMA] * (NBUF + NBUF + NIX + NBUF)
    ),
)


NMID = N_PAD // NW
BRM = 80


def _mid_body(acc_hbm, g1_hbm, dis_hbm, b1_hbm, g2_hbm, a0, a1, gv, db, bv,
              sem):
    c = lax.axis_index("c")
    s = lax.axis_index("s")
    w = c * NS + s
    r0 = w * NMID

    pltpu.sync_copy(b1_hbm, bv)
    pltpu.sync_copy(dis_hbm.at[pl.ds(r0, NMID)], db)

    def blk(t, carry):
        rb = r0 + t * BRM
        pltpu.sync_copy(acc_hbm.at[0, pl.ds(rb, BRM)], a0)
        pltpu.sync_copy(acc_hbm.at[1, pl.ds(rb, BRM)], a1)
        pltpu.sync_copy(g1_hbm.at[pl.ds(rb, BRM)], gv)

        @plsc.parallel_loop(0, BRM, unroll=2)
        def _(r):
            bc = plsc.load_gather(db, [jnp.full((L,), t * BRM + r, jnp.int32)])
            for q in range(HID // L):
                ds_ = pl.ds(q * L, L)
                val = a0[r, ds_] + a1[r, ds_] + gv[r, ds_]
                h = jnp.maximum(val * bc + bv[ds_], 0.0)
                a0[r, ds_] = h * bc

        pltpu.sync_copy(a0, g2_hbm.at[pl.ds(rb, BRM)])
        return carry

    lax.fori_loop(0, NMID // BRM, blk, 0)


_mid_call = pl.kernel(
    _mid_body,
    out_type=jax.ShapeDtypeStruct((N_PAD, HID), jnp.float32),
    mesh=_mesh,
    compiler_params=_sc_params,
    scratch_types=[
        pltpu.VMEM((BRM, HID), jnp.float32),
        pltpu.VMEM((BRM, HID), jnp.float32),
        pltpu.VMEM((BRM, HID), jnp.float32),
        pltpu.VMEM((NMID,), jnp.float32),
        pltpu.VMEM((HID,), jnp.float32),
        pltpu.SemaphoreType.DMA,
    ],
)


def _tc_pre_body(deg_ref, x_ref, w1_ref, g1_ref, dis_ref):
    deg = deg_ref[:, 0:1] + deg_ref[:, 1:2] + 1.0
    dis = lax.rsqrt(deg)
    h = jnp.dot(x_ref[...], w1_ref[...], preferred_element_type=jnp.float32)
    g1_ref[...] = h * dis
    dis_ref[...] = dis


_tc_pre = pl.pallas_call(
    _tc_pre_body,
    out_shape=(jax.ShapeDtypeStruct((N_PAD, HID), jnp.float32),
               jax.ShapeDtypeStruct((N_PAD, 1), jnp.float32)),
)


def _tc_mid_body(acc_ref, g1_ref, dis_ref, b1_ref, g2_ref):
    a = acc_ref[0] + acc_ref[1] + g1_ref[...]
    h = jnp.maximum(a * dis_ref[...] + b1_ref[...], 0.0)
    g2_ref[...] = h * dis_ref[...]


_tc_mid = pl.pallas_call(
    _tc_mid_body,
    out_shape=jax.ShapeDtypeStruct((N_PAD, HID), jnp.float32),
)


def _tc_post_body(acc_ref, g2_ref, dis_ref, w2_ref, b2_ref, batch_ref, out_ref):
    a = (acc_ref[0] + acc_ref[1] + g2_ref[...]) * dis_ref[...]
    y = jnp.dot(a, w2_ref[...], preferred_element_type=jnp.float32)
    y = jnp.maximum(y + b2_ref[...], 0.0)
    gid = lax.broadcasted_iota(jnp.int32, (G, N_PAD), 0)
    onehot = (batch_ref[...] == gid).astype(jnp.float32)
    sums = jnp.dot(onehot, y, preferred_element_type=jnp.float32)
    cnts = jnp.sum(onehot, axis=1, keepdims=True)
    out_ref[...] = sums / jnp.maximum(cnts, 1.0)


_tc_post = pl.pallas_call(
    _tc_post_body,
    out_shape=jax.ShapeDtypeStruct((G, OUT_DIM), jnp.float32),
)


def kernel(x, edge_index, edge_weight, batch_vec, W1, b1, W2, b2):
    sd2 = jnp.pad(edge_index.astype(jnp.int32), ((0, 0), (0, E_PAD - E)))
    dst2 = sd2[1].reshape(E_PAD // C, C)
    ew = jnp.pad(edge_weight.astype(jnp.float32), (0, E_PAD - E))
    ew2 = ew.reshape(E_PAD // C, C)

    x_p = jnp.pad(x, ((0, N_PAD - N), (0, 0)))
    batch_p = jnp.pad(batch_vec.astype(jnp.int32), (0, N_PAD - N),
                      constant_values=G).reshape(1, N_PAD)
    zeros_nd = jnp.zeros((N_PAD, HID), jnp.float32)

    deg2 = _deg_call(dst2, ew2)
    g1, dis = _tc_pre(deg2.T, x_p, W1)
    acc1 = _agg_call(g1, sd2, ew, zeros_nd)
    g2 = _mid_call(acc1, g1, dis.reshape(N_PAD), b1)
    acc2 = _agg_call(g2, sd2, ew, zeros_nd)
    return _tc_post(acc2, g2, dis, W2, b2.reshape(1, OUT_DIM), batch_p)
